# Initial kernel scaffold; baseline (speedup 1.0000x reference)
import jax
import jax.numpy as jnp
from jax.experimental import pallas as pl


def kernel(gru_input, edge_index_batch, edge_attr_batch, batch, w_ih, w_hh, b_ih, b_hh, w_gcn):
    raise NotImplementedError("write your pallas kernel here")


# speedup vs baseline: 2.9241x; 2.9241x over previous
import jax
import jax.numpy as jnp
from jax.experimental import pallas as pl
from jax.experimental.pallas import tpu as pltpu

B = 100
T = 100
D = 128
H = 128
N = B * T


def _gru_gcnproj_body(xs_ref, wih_ref, whh_ref, bih_ref, bhh_ref, wg_ref,
                      out_ref, h_ref):
    h_ref[...] = jnp.zeros((B, H), dtype=jnp.float32)

    def step(t, carry):
        x_t = xs_ref[t]
        gi = jnp.dot(x_t, wih_ref[...], preferred_element_type=jnp.float32) + bih_ref[...]
        gh = jnp.dot(h_ref[...], whh_ref[...], preferred_element_type=jnp.float32) + bhh_ref[...]
        i_r = gi[:, 0:H]
        i_z = gi[:, H:2 * H]
        i_n = gi[:, 2 * H:3 * H]
        h_r = gh[:, 0:H]
        h_z = gh[:, H:2 * H]
        h_n = gh[:, 2 * H:3 * H]
        r = jax.nn.sigmoid(i_r + h_r)
        z = jax.nn.sigmoid(i_z + h_z)
        n = jnp.tanh(i_n + r * h_n)
        h_new = (1.0 - z) * n + z * h_ref[...]
        h_ref[...] = h_new
        out_ref[t] = jnp.dot(h_new, wg_ref[...], preferred_element_type=jnp.float32)
        return carry

    jax.lax.fori_loop(0, T, step, 0)


def _gru_gcnproj(xs, w_ih, w_hh, b_ih, b_hh, w_gcn):
    return pl.pallas_call(
        _gru_gcnproj_body,
        out_shape=jax.ShapeDtypeStruct((T, B, H), jnp.float32),
        scratch_shapes=[pltpu.VMEM((B, H), jnp.float32)],
    )(xs, w_ih.T, w_hh.T, b_ih.reshape(1, 3 * H), b_hh.reshape(1, 3 * H), w_gcn.T)


def kernel(gru_input, edge_index_batch, edge_attr_batch, batch, w_ih, w_hh,
           b_ih, b_hh, w_gcn):
    xs = jnp.swapaxes(gru_input, 0, 1)
    xw_tb = _gru_gcnproj(xs, w_ih, w_hh, b_ih, b_hh, w_gcn)
    xw = jnp.swapaxes(xw_tb, 0, 1).reshape(N, H)

    src = edge_index_batch[0]
    dst = edge_index_batch[1]
    ew = edge_attr_batch
    deg = jax.ops.segment_sum(ew, dst, num_segments=N) + 1.0
    dinv = jax.lax.rsqrt(jnp.clip(deg, 1e-12))
    y = dinv[:, None] * xw
    msgs = ew[:, None] * jnp.take(y, src, axis=0)
    s = jax.ops.segment_sum(msgs, dst, num_segments=N)
    out = dinv[:, None] * (s + y)
    return out.reshape(B, T, H)


# speedup vs baseline: 11.1823x; 3.8242x over previous
import functools

import jax
import jax.numpy as jnp
from jax import lax
from jax.experimental import pallas as pl
from jax.experimental.pallas import tpu as pltpu
from jax.experimental.pallas import tpu_sc as plsc

B = 100
T = 100
D = 128
H = 128
N = B * T

NC = 2
NS = 16
NTILES = NC * NS
CHUNK = 128
CPT = 80
EPT = CPT * CHUNK
E_PAD = NTILES * EPT
NPAD = 10240
SLICE = NPAD // NS

_MESH = plsc.VectorSubcoreMesh(core_axis_name="c", subcore_axis_name="s")


def _gru_body(xs_ref, wih_ref, whh_ref, bih_ref, bhh_ref, wg_ref,
              out_ref, h_ref):
    h_ref[...] = jnp.zeros((B, H), dtype=jnp.float32)

    def step(t, carry):
        x_t = xs_ref[t]
        gi = jnp.dot(x_t, wih_ref[...], preferred_element_type=jnp.float32) + bih_ref[...]
        gh = jnp.dot(h_ref[...], whh_ref[...], preferred_element_type=jnp.float32) + bhh_ref[...]
        r = jax.nn.sigmoid(gi[:, 0:H] + gh[:, 0:H])
        z = jax.nn.sigmoid(gi[:, H:2 * H] + gh[:, H:2 * H])
        n = jnp.tanh(gi[:, 2 * H:3 * H] + r * gh[:, 2 * H:3 * H])
        h_new = (1.0 - z) * n + z * h_ref[...]
        h_ref[...] = h_new
        out_ref[t] = jnp.dot(h_new, wg_ref[...], preferred_element_type=jnp.float32)
        return carry

    lax.fori_loop(0, T, step, 0)


def _gru_gcnproj(xs, w_ih, w_hh, b_ih, b_hh, w_gcn):
    return pl.pallas_call(
        _gru_body,
        out_shape=jax.ShapeDtypeStruct((T, B, H), jnp.float32),
        scratch_shapes=[pltpu.VMEM((B, H), jnp.float32)],
    )(xs, w_ih.T, w_hh.T, b_ih.reshape(1, 3 * H), b_hh.reshape(1, 3 * H), w_gcn.T)


def _deg_body(dst_hbm, ew_hbm, deg_out, dbuf, ebuf, zbuf, deg_sh):
    c = lax.axis_index("c")
    s = lax.axis_index("s")
    wid = c * NS + s
    pltpu.sync_copy(dst_hbm.at[pl.ds(wid * CPT, CPT)], dbuf)
    pltpu.sync_copy(ew_hbm.at[pl.ds(wid * CPT, CPT)], ebuf)

    def zrow(i, carry):
        zbuf[pl.ds(i * 16, 16)] = jnp.zeros((16,), jnp.float32)
        return carry

    lax.fori_loop(0, SLICE // 16, zrow, 0)
    pltpu.sync_copy(zbuf, deg_sh.at[pl.ds(s * SLICE, SLICE)])
    plsc.subcore_barrier()

    def chunk(j, carry):
        pltpu.sync_copy(ebuf.at[j], deg_sh.at[dbuf.at[j]], add=True)
        return carry

    lax.fori_loop(0, CPT, chunk, 0)
    plsc.subcore_barrier()
    pltpu.sync_copy(deg_sh.at[pl.ds(s * SLICE, SLICE)],
                    deg_out.at[c, pl.ds(s * SLICE, SLICE)])


_deg_kernel = pl.kernel(
    _deg_body,
    out_type=jax.ShapeDtypeStruct((NC, NPAD), jnp.float32),
    mesh=_MESH,
    scratch_types=[
        pltpu.VMEM((CPT, CHUNK), jnp.int32),
        pltpu.VMEM((CPT, CHUNK), jnp.float32),
        pltpu.VMEM((SLICE,), jnp.float32),
        pltpu.VMEM_SHARED((NPAD,), jnp.float32),
    ],
)


def _scat_body(src_hbm, dst_hbm, ew_hbm, y_hbm, s_out,
               sbuf, dbuf, ebuf, rows, s_sh, sem):
    c = lax.axis_index("c")
    s = lax.axis_index("s")
    wid = c * NS + s

    def zrow(i, carry):
        for q in range(8):
            rows[i, pl.ds(q * 16, 16)] = jnp.zeros((16,), jnp.float32)
        return carry

    lax.fori_loop(0, CHUNK, zrow, 0)

    def zcopy(q, carry):
        pltpu.sync_copy(rows, s_sh.at[pl.ds(s * SLICE + q * CHUNK, CHUNK)])
        return carry

    lax.fori_loop(0, SLICE // CHUNK, zcopy, 0)

    plsc.subcore_barrier()

    for p in range(2):
        base = wid * CPT + p * (CPT // 2)
        pltpu.sync_copy(src_hbm.at[pl.ds(base, CPT // 2)], sbuf)
        pltpu.sync_copy(dst_hbm.at[pl.ds(base, CPT // 2)], dbuf)
        pltpu.sync_copy(ew_hbm.at[pl.ds(base, CPT // 2)], ebuf)

        def chunk(j, carry):
            pltpu.async_copy(y_hbm.at[sbuf.at[j]], rows, sem).wait()

            def scale(k, carry2):
                cv = ebuf[j, pl.ds(k * 16, 16)]
                for l in range(16):
                    coef = cv[l]
                    i = k * 16 + l
                    for q in range(8):
                        rows[i, pl.ds(q * 16, 16)] = rows[i, pl.ds(q * 16, 16)] * coef
                return carry2

            lax.fori_loop(0, CHUNK // 16, scale, 0)
            pltpu.sync_copy(rows, s_sh.at[dbuf.at[j]], add=True)
            return carry

        lax.fori_loop(0, CPT // 2, chunk, 0)

    plsc.subcore_barrier()
    pltpu.sync_copy(s_sh.at[pl.ds(s * SLICE, SLICE)],
                    s_out.at[c, pl.ds(s * SLICE, SLICE)])


_scat_kernel = pl.kernel(
    _scat_body,
    out_type=jax.ShapeDtypeStruct((NC, NPAD, H), jnp.float32),
    mesh=_MESH,
    scratch_types=[
        pltpu.VMEM((CPT // 2, CHUNK), jnp.int32),
        pltpu.VMEM((CPT // 2, CHUNK), jnp.int32),
        pltpu.VMEM((CPT // 2, CHUNK), jnp.float32),
        pltpu.VMEM((CHUNK, H), jnp.float32),
        pltpu.VMEM_SHARED((NPAD, H), jnp.float32),
        pltpu.SemaphoreType.DMA,
    ],
)


def _dinv_body(deg_ref, out_ref):
    d = deg_ref[0] + deg_ref[1] + 1.0
    out_ref[...] = lax.rsqrt(jnp.maximum(d, 1e-12))


def _dinv(deg):
    out = pl.pallas_call(
        _dinv_body,
        out_shape=jax.ShapeDtypeStruct((NPAD // 128, 128), jnp.float32),
    )(deg.reshape(NC, NPAD // 128, 128))
    return out.reshape(NPAD)


def _y_body(dinv_ref, xw_ref, out_ref):
    out_ref[...] = dinv_ref[:N, :] * xw_ref[...]


def _y_scale(dinv, xw):
    return pl.pallas_call(
        _y_body,
        out_shape=jax.ShapeDtypeStruct((N, H), jnp.float32),
    )(dinv.reshape(NPAD, 1), xw)


def _final_body(s_ref, dinv_ref, y_ref, out_ref):
    dinv = dinv_ref[:N, :]
    out_ref[...] = dinv * (s_ref[0, :N, :] + s_ref[1, :N, :] + y_ref[...])


def _final(s_parts, dinv, y):
    return pl.pallas_call(
        _final_body,
        out_shape=jax.ShapeDtypeStruct((N, H), jnp.float32),
    )(s_parts, dinv.reshape(NPAD, 1), y)


def kernel(gru_input, edge_index_batch, edge_attr_batch, batch, w_ih, w_hh,
           b_ih, b_hh, w_gcn):
    xs = jnp.swapaxes(gru_input, 0, 1)
    xw_tb = _gru_gcnproj(xs, w_ih, w_hh, b_ih, b_hh, w_gcn)
    xw = jnp.swapaxes(xw_tb, 0, 1).reshape(N, H)

    pad = E_PAD - edge_index_batch.shape[1]
    src = jnp.pad(edge_index_batch[0].astype(jnp.int32), (0, pad)).reshape(
        NTILES * CPT, CHUNK)
    dst = jnp.pad(edge_index_batch[1].astype(jnp.int32), (0, pad)).reshape(
        NTILES * CPT, CHUNK)
    ew = jnp.pad(edge_attr_batch, (0, pad)).reshape(NTILES * CPT, CHUNK)

    deg = _deg_kernel(dst, ew)
    dinv = _dinv(deg)
    y = _y_scale(dinv, xw)
    s_parts = _scat_kernel(src, dst, ew, y)
    out = _final(s_parts, dinv, y)
    return out.reshape(B, T, H)


# speedup vs baseline: 13.3101x; 1.1903x over previous
import functools

import jax
import jax.numpy as jnp
from jax import lax
from jax.experimental import pallas as pl
from jax.experimental.pallas import tpu as pltpu
from jax.experimental.pallas import tpu_sc as plsc

B = 100
T = 100
D = 128
H = 128
N = B * T

NC = 2
NS = 16
NTILES = NC * NS
CHUNK = 128
CPT = 80
SPC = 40
EPT = CPT * CHUNK
E_PAD = NTILES * EPT
NPAD = 10240
SLICE = NPAD // NS

_MESH = plsc.VectorSubcoreMesh(core_axis_name="c", subcore_axis_name="s")


def _gru_body(xs_ref, wih_ref, whh_ref, bih_ref, bhh_ref, wg_ref,
              out_ref, h_ref):
    h_ref[...] = jnp.zeros((B, H), dtype=jnp.float32)

    def step(t, carry):
        x_t = xs_ref[t]
        gi = jnp.dot(x_t, wih_ref[...], preferred_element_type=jnp.float32) + bih_ref[...]
        gh = jnp.dot(h_ref[...], whh_ref[...], preferred_element_type=jnp.float32) + bhh_ref[...]
        r = jax.nn.sigmoid(gi[:, 0:H] + gh[:, 0:H])
        z = jax.nn.sigmoid(gi[:, H:2 * H] + gh[:, H:2 * H])
        n = jnp.tanh(gi[:, 2 * H:3 * H] + r * gh[:, 2 * H:3 * H])
        h_new = (1.0 - z) * n + z * h_ref[...]
        h_ref[...] = h_new
        out_ref[t] = jnp.dot(h_new, wg_ref[...], preferred_element_type=jnp.float32)
        return carry

    lax.fori_loop(0, T, step, 0)


def _gru_gcnproj(xs, w_ih, w_hh, b_ih, b_hh, w_gcn):
    return pl.pallas_call(
        _gru_body,
        out_shape=jax.ShapeDtypeStruct((T, B, H), jnp.float32),
        scratch_shapes=[pltpu.VMEM((B, H), jnp.float32)],
    )(xs, w_ih.T, w_hh.T, b_ih.reshape(1, 3 * H), b_hh.reshape(1, 3 * H), w_gcn.T)


def _deg_body(dst_hbm, ew_hbm, deg_out, dbuf, ebuf, zbuf, deg_sh):
    c = lax.axis_index("c")
    s = lax.axis_index("s")
    wid = c * NS + s
    pltpu.sync_copy(dst_hbm.at[pl.ds(wid * CPT, CPT)], dbuf)
    pltpu.sync_copy(ew_hbm.at[pl.ds(wid * CPT, CPT)], ebuf)

    def zrow(i, carry):
        zbuf[pl.ds(i * 16, 16)] = jnp.zeros((16,), jnp.float32)
        return carry

    lax.fori_loop(0, SLICE // 16, zrow, 0)
    pltpu.sync_copy(zbuf, deg_sh.at[pl.ds(s * SLICE, SLICE)])
    plsc.subcore_barrier()

    def chunk(j, carry):
        pltpu.sync_copy(ebuf.at[j], deg_sh.at[dbuf.at[j]], add=True)
        return carry

    lax.fori_loop(0, CPT, chunk, 0)
    plsc.subcore_barrier()
    pltpu.sync_copy(deg_sh.at[pl.ds(s * SLICE, SLICE)],
                    deg_out.at[c, pl.ds(s * SLICE, SLICE)])


_deg_kernel = pl.kernel(
    _deg_body,
    out_type=jax.ShapeDtypeStruct((NC, NPAD), jnp.float32),
    mesh=_MESH,
    scratch_types=[
        pltpu.VMEM((CPT, CHUNK), jnp.int32),
        pltpu.VMEM((CPT, CHUNK), jnp.float32),
        pltpu.VMEM((SLICE,), jnp.float32),
        pltpu.VMEM_SHARED((NPAD,), jnp.float32),
    ],
)


def _scat_body(src_hbm, dst_hbm, ew_hbm, y_hbm, s_out,
               sbuf, dbuf, ebuf, rows0, rows1, s_sh, sem0, sem1):
    c = lax.axis_index("c")
    s = lax.axis_index("s")
    wid = c * NS + s

    def zrow(i, carry):
        for q in range(8):
            rows0[i, pl.ds(q * 16, 16)] = jnp.zeros((16,), jnp.float32)
        return carry

    lax.fori_loop(0, CHUNK, zrow, 0)

    def zcopy(q, carry):
        pltpu.sync_copy(rows0, s_sh.at[pl.ds(s * SLICE + q * CHUNK, CHUNK)])
        return carry

    lax.fori_loop(0, SLICE // CHUNK, zcopy, 0)

    plsc.subcore_barrier()

    nstage = CPT // SPC
    for p in range(nstage):
        base = wid * CPT + p * SPC
        pltpu.sync_copy(src_hbm.at[pl.ds(base, SPC)], sbuf)
        pltpu.sync_copy(dst_hbm.at[pl.ds(base, SPC)], dbuf)
        pltpu.sync_copy(ew_hbm.at[pl.ds(base, SPC)], ebuf)

        pltpu.make_async_copy(y_hbm.at[sbuf.at[0]], rows0, sem0).start()

        def process(j, rows_b, sem_b, rows_n, sem_n):
            @pl.when(j < SPC - 1)
            def _():
                pltpu.make_async_copy(y_hbm.at[sbuf.at[j + 1]], rows_n,
                                      sem_n).start()

            pltpu.make_async_copy(y_hbm.at[sbuf.at[j]], rows_b, sem_b).wait()

            def scale(k, carry2):
                cv = ebuf[j, pl.ds(k * 16, 16)]
                for l in range(16):
                    coef = cv[l]
                    i = k * 16 + l
                    for q in range(8):
                        rows_b[i, pl.ds(q * 16, 16)] = (
                            rows_b[i, pl.ds(q * 16, 16)] * coef)
                return carry2

            lax.fori_loop(0, CHUNK // 16, scale, 0)
            pltpu.sync_copy(rows_b, s_sh.at[dbuf.at[j]], add=True)

        def pair(jj, carry):
            j = jj * 2
            process(j, rows0, sem0, rows1, sem1)
            process(j + 1, rows1, sem1, rows0, sem0)
            return carry

        lax.fori_loop(0, SPC // 2, pair, 0)

    plsc.subcore_barrier()
    pltpu.sync_copy(s_sh.at[pl.ds(s * SLICE, SLICE)],
                    s_out.at[c, pl.ds(s * SLICE, SLICE)])


_scat_kernel = pl.kernel(
    _scat_body,
    out_type=jax.ShapeDtypeStruct((NC, NPAD, H), jnp.float32),
    mesh=_MESH,
    scratch_types=[
        pltpu.VMEM((SPC, CHUNK), jnp.int32),
        pltpu.VMEM((SPC, CHUNK), jnp.int32),
        pltpu.VMEM((SPC, CHUNK), jnp.float32),
        pltpu.VMEM((CHUNK, H), jnp.float32),
        pltpu.VMEM((CHUNK, H), jnp.float32),
        pltpu.VMEM_SHARED((NPAD, H), jnp.float32),
        pltpu.SemaphoreType.DMA,
        pltpu.SemaphoreType.DMA,
    ],
)


def _dinv_body(deg_ref, out_ref):
    d = deg_ref[0] + deg_ref[1] + 1.0
    out_ref[...] = lax.rsqrt(jnp.maximum(d, 1e-12))


def _dinv(deg):
    out = pl.pallas_call(
        _dinv_body,
        out_shape=jax.ShapeDtypeStruct((NPAD // 128, 128), jnp.float32),
    )(deg.reshape(NC, NPAD // 128, 128))
    return out.reshape(NPAD)


def _y_body(dinv_ref, xw_ref, out_ref):
    out_ref[...] = dinv_ref[:N, :] * xw_ref[...]


def _y_scale(dinv, xw):
    return pl.pallas_call(
        _y_body,
        out_shape=jax.ShapeDtypeStruct((N, H), jnp.float32),
    )(dinv.reshape(NPAD, 1), xw)


def _final_body(s_ref, dinv_ref, y_ref, out_ref):
    dinv = dinv_ref[:N, :]
    out_ref[...] = dinv * (s_ref[0, :N, :] + s_ref[1, :N, :] + y_ref[...])


def _final(s_parts, dinv, y):
    return pl.pallas_call(
        _final_body,
        out_shape=jax.ShapeDtypeStruct((N, H), jnp.float32),
    )(s_parts, dinv.reshape(NPAD, 1), y)


def kernel(gru_input, edge_index_batch, edge_attr_batch, batch, w_ih, w_hh,
           b_ih, b_hh, w_gcn):
    xs = jnp.swapaxes(gru_input, 0, 1)
    xw_tb = _gru_gcnproj(xs, w_ih, w_hh, b_ih, b_hh, w_gcn)
    xw = jnp.swapaxes(xw_tb, 0, 1).reshape(N, H)

    pad = E_PAD - edge_index_batch.shape[1]
    src = jnp.pad(edge_index_batch[0].astype(jnp.int32), (0, pad)).reshape(
        NTILES * CPT, CHUNK)
    dst = jnp.pad(edge_index_batch[1].astype(jnp.int32), (0, pad)).reshape(
        NTILES * CPT, CHUNK)
    ew = jnp.pad(edge_attr_batch, (0, pad)).reshape(NTILES * CPT, CHUNK)

    deg = _deg_kernel(dst, ew)
    dinv = _dinv(deg)
    y = _y_scale(dinv, xw)
    s_parts = _scat_kernel(src, dst, ew, y)
    out = _final(s_parts, dinv, y)
    return out.reshape(B, T, H)


# speedup vs baseline: 13.6816x; 1.0279x over previous
import functools

import jax
import jax.numpy as jnp
from jax import lax
from jax.experimental import pallas as pl
from jax.experimental.pallas import tpu as pltpu
from jax.experimental.pallas import tpu_sc as plsc

B = 100
T = 100
D = 128
H = 128
N = B * T

NC = 2
NS = 16
NTILES = NC * NS
CHUNK = 128
CPT = 80
SPC = 40
EPT = CPT * CHUNK
E_PAD = NTILES * EPT
NPAD = 10240
SLICE = NPAD // NS

_MESH = plsc.VectorSubcoreMesh(core_axis_name="c", subcore_axis_name="s")


def _gru_body(xs_ref, wih_ref, whh_ref, bih_ref, bhh_ref,
              out_ref, h_ref, gi_ref):
    gi_ref[...] = (jnp.dot(
        xs_ref[...].reshape(T * B, D), wih_ref[...],
        preferred_element_type=jnp.float32) + bih_ref[...]).reshape(
            T, B, 3 * H)
    h_ref[...] = jnp.zeros((B, H), dtype=jnp.float32)

    def step(t, carry):
        gi = gi_ref[t]
        gh = jnp.dot(h_ref[...], whh_ref[...], preferred_element_type=jnp.float32) + bhh_ref[...]
        r = jax.nn.sigmoid(gi[:, 0:H] + gh[:, 0:H])
        z = jax.nn.sigmoid(gi[:, H:2 * H] + gh[:, H:2 * H])
        n = jnp.tanh(gi[:, 2 * H:3 * H] + r * gh[:, 2 * H:3 * H])
        h_new = (1.0 - z) * n + z * h_ref[...]
        h_ref[...] = h_new
        out_ref[t] = h_new
        return carry

    lax.fori_loop(0, T, step, 0)


def _gru(xs, w_ih, w_hh, b_ih, b_hh):
    return pl.pallas_call(
        _gru_body,
        out_shape=jax.ShapeDtypeStruct((T, B, H), jnp.float32),
        scratch_shapes=[pltpu.VMEM((B, H), jnp.float32),
                        pltpu.VMEM((T, B, 3 * H), jnp.float32)],
    )(xs, w_ih.T, w_hh.T, b_ih.reshape(1, 3 * H), b_hh.reshape(1, 3 * H))


def _deg_body(dst_hbm, ew_hbm, deg_out, dbuf, ebuf, zbuf, deg_sh):
    c = lax.axis_index("c")
    s = lax.axis_index("s")
    wid = c * NS + s
    pltpu.sync_copy(dst_hbm.at[pl.ds(wid * CPT, CPT)], dbuf)
    pltpu.sync_copy(ew_hbm.at[pl.ds(wid * CPT, CPT)], ebuf)

    def zrow(i, carry):
        zbuf[pl.ds(i * 16, 16)] = jnp.zeros((16,), jnp.float32)
        return carry

    lax.fori_loop(0, SLICE // 16, zrow, 0)
    pltpu.sync_copy(zbuf, deg_sh.at[pl.ds(s * SLICE, SLICE)])
    plsc.subcore_barrier()

    def chunk(j, carry):
        pltpu.sync_copy(ebuf.at[j], deg_sh.at[dbuf.at[j]], add=True)
        return carry

    lax.fori_loop(0, CPT, chunk, 0)
    plsc.subcore_barrier()
    pltpu.sync_copy(deg_sh.at[pl.ds(s * SLICE, SLICE)],
                    deg_out.at[c, pl.ds(s * SLICE, SLICE)])


_deg_kernel = pl.kernel(
    _deg_body,
    out_type=jax.ShapeDtypeStruct((NC, NPAD), jnp.float32),
    mesh=_MESH,
    scratch_types=[
        pltpu.VMEM((CPT, CHUNK), jnp.int32),
        pltpu.VMEM((CPT, CHUNK), jnp.float32),
        pltpu.VMEM((SLICE,), jnp.float32),
        pltpu.VMEM_SHARED((NPAD,), jnp.float32),
    ],
)


def _scat_body(src_hbm, dst_hbm, ew_hbm, y_hbm, s_out,
               sbuf, dbuf, ebuf, rows0, rows1, s_sh, sem0, sem1):
    c = lax.axis_index("c")
    s = lax.axis_index("s")
    wid = c * NS + s

    def zrow(i, carry):
        for q in range(8):
            rows0[i, pl.ds(q * 16, 16)] = jnp.zeros((16,), jnp.float32)
        return carry

    lax.fori_loop(0, CHUNK, zrow, 0)

    def zcopy(q, carry):
        pltpu.sync_copy(rows0, s_sh.at[pl.ds(s * SLICE + q * CHUNK, CHUNK)])
        return carry

    lax.fori_loop(0, SLICE // CHUNK, zcopy, 0)

    plsc.subcore_barrier()

    nstage = CPT // SPC
    for p in range(nstage):
        base = wid * CPT + p * SPC
        pltpu.sync_copy(src_hbm.at[pl.ds(base, SPC)], sbuf)
        pltpu.sync_copy(dst_hbm.at[pl.ds(base, SPC)], dbuf)
        pltpu.sync_copy(ew_hbm.at[pl.ds(base, SPC)], ebuf)

        pltpu.make_async_copy(y_hbm.at[sbuf.at[0]], rows0, sem0).start()

        def process(j, rows_b, sem_b, rows_n, sem_n):
            @pl.when(j < SPC - 1)
            def _():
                pltpu.make_async_copy(y_hbm.at[sbuf.at[j + 1]], rows_n,
                                      sem_n).start()

            pltpu.make_async_copy(y_hbm.at[sbuf.at[j]], rows_b, sem_b).wait()

            def scale(k, carry2):
                cv = ebuf[j, pl.ds(k * 16, 16)]
                for l in range(16):
                    coef = cv[l]
                    i = k * 16 + l
                    for q in range(8):
                        rows_b[i, pl.ds(q * 16, 16)] = (
                            rows_b[i, pl.ds(q * 16, 16)] * coef)
                return carry2

            lax.fori_loop(0, CHUNK // 16, scale, 0)
            pltpu.sync_copy(rows_b, s_sh.at[dbuf.at[j]], add=True)

        def pair(jj, carry):
            j = jj * 2
            process(j, rows0, sem0, rows1, sem1)
            process(j + 1, rows1, sem1, rows0, sem0)
            return carry

        lax.fori_loop(0, SPC // 2, pair, 0)

    plsc.subcore_barrier()
    pltpu.sync_copy(s_sh.at[pl.ds(s * SLICE, SLICE)],
                    s_out.at[c, pl.ds(s * SLICE, SLICE)])


_scat_kernel = pl.kernel(
    _scat_body,
    out_type=jax.ShapeDtypeStruct((NC, NPAD, H), jnp.float32),
    mesh=_MESH,
    scratch_types=[
        pltpu.VMEM((SPC, CHUNK), jnp.int32),
        pltpu.VMEM((SPC, CHUNK), jnp.int32),
        pltpu.VMEM((SPC, CHUNK), jnp.float32),
        pltpu.VMEM((CHUNK, H), jnp.float32),
        pltpu.VMEM((CHUNK, H), jnp.float32),
        pltpu.VMEM_SHARED((NPAD, H), jnp.float32),
        pltpu.SemaphoreType.DMA,
        pltpu.SemaphoreType.DMA,
    ],
)


def _dinv_body(deg_ref, out_ref):
    d = deg_ref[0] + deg_ref[1] + 1.0
    out_ref[...] = lax.rsqrt(jnp.maximum(d, 1e-12))


def _dinv(deg):
    out = pl.pallas_call(
        _dinv_body,
        out_shape=jax.ShapeDtypeStruct((NPAD // 128, 128), jnp.float32),
    )(deg.reshape(NC, NPAD // 128, 128))
    return out.reshape(NPAD)


def _y_body(dinv_ref, xw_ref, out_ref):
    out_ref[...] = dinv_ref[:N, :] * xw_ref[...]


def _y_scale(dinv, xw):
    return pl.pallas_call(
        _y_body,
        out_shape=jax.ShapeDtypeStruct((N, H), jnp.float32),
    )(dinv.reshape(NPAD, 1), xw)


def _final_body(s_ref, dinv_ref, y_ref, wg_ref, out_ref):
    dinv = dinv_ref[:N, :]
    z = dinv * (s_ref[0, :N, :] + s_ref[1, :N, :] + y_ref[...])
    out_ref[...] = jnp.dot(z, wg_ref[...], preferred_element_type=jnp.float32)


def _final(s_parts, dinv, y, w_gcn):
    return pl.pallas_call(
        _final_body,
        out_shape=jax.ShapeDtypeStruct((N, H), jnp.float32),
    )(s_parts, dinv.reshape(NPAD, 1), y, w_gcn.T)


def kernel(gru_input, edge_index_batch, edge_attr_batch, batch, w_ih, w_hh,
           b_ih, b_hh, w_gcn):
    xs = jnp.swapaxes(gru_input, 0, 1)
    h_tb = _gru(xs, w_ih, w_hh, b_ih, b_hh)
    hmat = jnp.swapaxes(h_tb, 0, 1).reshape(N, H)

    pad = E_PAD - edge_index_batch.shape[1]
    src = jnp.pad(edge_index_batch[0].astype(jnp.int32), (0, pad)).reshape(
        NTILES * CPT, CHUNK)
    dst = jnp.pad(edge_index_batch[1].astype(jnp.int32), (0, pad)).reshape(
        NTILES * CPT, CHUNK)
    ew = jnp.pad(edge_attr_batch, (0, pad)).reshape(NTILES * CPT, CHUNK)

    deg = _deg_kernel(dst, ew)
    dinv = _dinv(deg)
    y = _y_scale(dinv, hmat)
    s_parts = _scat_kernel(src, dst, ew, y)
    out = _final(s_parts, dinv, y, w_gcn)
    return out.reshape(B, T, H)


# speedup vs baseline: 29.6874x; 2.1699x over previous
import functools

import jax
import jax.numpy as jnp
from jax import lax
from jax.experimental import pallas as pl
from jax.experimental.pallas import tpu as pltpu
from jax.experimental.pallas import tpu_sc as plsc

B = 100
T = 100
D = 128
H = 128
N = B * T

NC = 2
NS = 16
NTILES = NC * NS
CHUNK = 128
CPT = 80
SPC = 40
EPT = CPT * CHUNK
E_PAD = NTILES * EPT
NPAD = 10240
SLICE = NPAD // NS

_MESH = plsc.VectorSubcoreMesh(core_axis_name="c", subcore_axis_name="s")


def _gru_body(xs_ref, wih_ref, whh_ref, bih_ref, bhh_ref,
              out_ref, h_ref, gi_ref):
    gi_ref[...] = (jnp.dot(
        xs_ref[...].reshape(T * B, D), wih_ref[...],
        preferred_element_type=jnp.float32) + bih_ref[...]).reshape(
            T, B, 3 * H)
    h_ref[...] = jnp.zeros((B, H), dtype=jnp.float32)

    def step(t, carry):
        gi = gi_ref[t]
        gh = jnp.dot(h_ref[...], whh_ref[...], preferred_element_type=jnp.float32) + bhh_ref[...]
        r = jax.nn.sigmoid(gi[:, 0:H] + gh[:, 0:H])
        z = jax.nn.sigmoid(gi[:, H:2 * H] + gh[:, H:2 * H])
        n = jnp.tanh(gi[:, 2 * H:3 * H] + r * gh[:, 2 * H:3 * H])
        h_new = (1.0 - z) * n + z * h_ref[...]
        h_ref[...] = h_new
        out_ref[t] = h_new
        return carry

    lax.fori_loop(0, T, step, 0)


def _gru(xs, w_ih, w_hh, b_ih, b_hh):
    return pl.pallas_call(
        _gru_body,
        out_shape=jax.ShapeDtypeStruct((T, B, H), jnp.float32),
        scratch_shapes=[pltpu.VMEM((B, H), jnp.float32),
                        pltpu.VMEM((T, B, 3 * H), jnp.float32)],
    )(xs, w_ih.T, w_hh.T, b_ih.reshape(1, 3 * H), b_hh.reshape(1, 3 * H))


def _deg_body(dst_hbm, ew_hbm, deg_out, dbuf, ebuf, zbuf, deg_sh):
    c = lax.axis_index("c")
    s = lax.axis_index("s")
    wid = c * NS + s
    pltpu.sync_copy(dst_hbm.at[pl.ds(wid * CPT, CPT)], dbuf)
    pltpu.sync_copy(ew_hbm.at[pl.ds(wid * CPT, CPT)], ebuf)

    def zrow(i, carry):
        zbuf[pl.ds(i * 16, 16)] = jnp.zeros((16,), jnp.float32)
        return carry

    lax.fori_loop(0, SLICE // 16, zrow, 0)
    pltpu.sync_copy(zbuf, deg_sh.at[pl.ds(s * SLICE, SLICE)])
    plsc.subcore_barrier()

    def chunk(j, carry):
        pltpu.sync_copy(ebuf.at[j], deg_sh.at[dbuf.at[j]], add=True)
        return carry

    lax.fori_loop(0, CPT, chunk, 0)
    plsc.subcore_barrier()
    pltpu.sync_copy(deg_sh.at[pl.ds(s * SLICE, SLICE)],
                    deg_out.at[c, pl.ds(s * SLICE, SLICE)])


_deg_kernel = pl.kernel(
    _deg_body,
    out_type=jax.ShapeDtypeStruct((NC, NPAD), jnp.float32),
    mesh=_MESH,
    scratch_types=[
        pltpu.VMEM((CPT, CHUNK), jnp.int32),
        pltpu.VMEM((CPT, CHUNK), jnp.float32),
        pltpu.VMEM((SLICE,), jnp.float32),
        pltpu.VMEM_SHARED((NPAD,), jnp.float32),
    ],
)


def _scat_body(src_hbm, dst_hbm, ew_hbm, y_hbm, s_out,
               sbuf, dbuf, ebuf, rows0, rows1, s_sh, sem0, sem1):
    c = lax.axis_index("c")
    s = lax.axis_index("s")
    wid = c * NS + s

    def zrow(i, carry):
        for q in range(8):
            rows0[i, pl.ds(q * 16, 16)] = jnp.zeros((16,), jnp.float32)
        return carry

    lax.fori_loop(0, CHUNK, zrow, 0)

    def zcopy(q, carry):
        pltpu.sync_copy(rows0, s_sh.at[pl.ds(s * SLICE + q * CHUNK, CHUNK)])
        return carry

    lax.fori_loop(0, SLICE // CHUNK, zcopy, 0)

    plsc.subcore_barrier()

    nstage = CPT // SPC
    for p in range(nstage):
        base = wid * CPT + p * SPC
        pltpu.sync_copy(src_hbm.at[pl.ds(base, SPC)], sbuf)
        pltpu.sync_copy(dst_hbm.at[pl.ds(base, SPC)], dbuf)
        pltpu.sync_copy(ew_hbm.at[pl.ds(base, SPC)], ebuf)

        pltpu.make_async_copy(y_hbm.at[sbuf.at[0]], rows0, sem0).start()

        def process(j, rows_b, sem_b, rows_n, sem_n):
            @pl.when(j < SPC - 1)
            def _():
                pltpu.make_async_copy(y_hbm.at[sbuf.at[j + 1]], rows_n,
                                      sem_n).start()

            pltpu.make_async_copy(y_hbm.at[sbuf.at[j]], rows_b, sem_b).wait()

            def scale(k, carry2):
                cv = ebuf[j, pl.ds(k * 16, 16)]
                for l in range(16):
                    coef = cv[l]
                    i = k * 16 + l
                    for q in range(8):
                        rows_b[i, pl.ds(q * 16, 16)] = (
                            rows_b[i, pl.ds(q * 16, 16)] * coef)
                return carry2

            lax.fori_loop(0, CHUNK // 16, scale, 0)
            pltpu.sync_copy(rows_b, s_sh.at[dbuf.at[j]], add=True)

        def pair(jj, carry):
            j = jj * 2
            process(j, rows0, sem0, rows1, sem1)
            process(j + 1, rows1, sem1, rows0, sem0)
            return carry

        lax.fori_loop(0, SPC // 2, pair, 0)

    plsc.subcore_barrier()
    pltpu.sync_copy(s_sh.at[pl.ds(s * SLICE, SLICE)],
                    s_out.at[c, pl.ds(s * SLICE, SLICE)])


_scat_kernel = pl.kernel(
    _scat_body,
    out_type=jax.ShapeDtypeStruct((NC, NPAD, H), jnp.float32),
    mesh=_MESH,
    scratch_types=[
        pltpu.VMEM((SPC, CHUNK), jnp.int32),
        pltpu.VMEM((SPC, CHUNK), jnp.int32),
        pltpu.VMEM((SPC, CHUNK), jnp.float32),
        pltpu.VMEM((CHUNK, H), jnp.float32),
        pltpu.VMEM((CHUNK, H), jnp.float32),
        pltpu.VMEM_SHARED((NPAD, H), jnp.float32),
        pltpu.SemaphoreType.DMA,
        pltpu.SemaphoreType.DMA,
    ],
)


def _dinv_body(deg_ref, out_ref):
    d = deg_ref[0] + deg_ref[1] + 1.0
    out_ref[...] = lax.rsqrt(jnp.maximum(d, 1e-12))


def _dinv(deg):
    out = pl.pallas_call(
        _dinv_body,
        out_shape=jax.ShapeDtypeStruct((NPAD // 128, 128), jnp.float32),
    )(deg.reshape(NC, NPAD // 128, 128))
    return out.reshape(NPAD)


def _y_body(dinv_ref, xw_ref, out_ref):
    out_ref[...] = dinv_ref[:N, :] * xw_ref[...]


def _y_scale(dinv, xw):
    return pl.pallas_call(
        _y_body,
        out_shape=jax.ShapeDtypeStruct((N, H), jnp.float32),
    )(dinv.reshape(NPAD, 1), xw)


def _final_body(s_ref, dinv_ref, y_ref, wg_ref, out_ref):
    dinv = dinv_ref[:N, :]
    z = dinv * (s_ref[0, :N, :] + s_ref[1, :N, :] + y_ref[...])
    out_ref[...] = jnp.dot(z, wg_ref[...], preferred_element_type=jnp.float32)


def _final(s_parts, dinv, y, w_gcn):
    return pl.pallas_call(
        _final_body,
        out_shape=jax.ShapeDtypeStruct((N, H), jnp.float32),
    )(s_parts, dinv.reshape(NPAD, 1), y, w_gcn.T)


def kernel(gru_input, edge_index_batch, edge_attr_batch, batch, w_ih, w_hh,
           b_ih, b_hh, w_gcn):
    xs = jnp.swapaxes(gru_input, 0, 1)
    h_tb = _gru(xs, w_ih, w_hh, b_ih, b_hh)
    hmat = jnp.swapaxes(h_tb, 0, 1).reshape(N, H)

    pad = E_PAD - edge_index_batch.shape[1]
    pidx = jnp.arange(pad, dtype=jnp.int32)
    src = jnp.concatenate(
        [edge_index_batch[0].astype(jnp.int32), pidx % N]).reshape(
            NTILES * CPT, CHUNK)
    dst = jnp.concatenate(
        [edge_index_batch[1].astype(jnp.int32), pidx % NPAD]).reshape(
            NTILES * CPT, CHUNK)
    ew = jnp.pad(edge_attr_batch, (0, pad)).reshape(NTILES * CPT, CHUNK)

    deg = _deg_kernel(dst, ew)
    dinv = _dinv(deg)
    y = _y_scale(dinv, hmat)
    s_parts = _scat_kernel(src, dst, ew, y)
    out = _final(s_parts, dinv, y, w_gcn)
    return out.reshape(B, T, H)
